# Initial kernel scaffold; baseline (speedup 1.0000x reference)
import jax
import jax.numpy as jnp
from jax.experimental import pallas as pl


def kernel(x, edge_index, W1, b1, W2, b2):
    raise NotImplementedError("write your pallas kernel here")


# speedup vs baseline: 15.8301x; 15.8301x over previous
import functools

import jax
import jax.numpy as jnp
from jax import lax
from jax.experimental import pallas as pl
from jax.experimental.pallas import tpu as pltpu
from jax.experimental.pallas import tpu_sc as plsc

N_NODES = 10000
D = 128
N_PAD = 10240
NC, NS = 2, 16
NW = NC * NS
CHUNK = 128
RPT = N_PAD // NS
BR = 512
GRID = N_PAD // BR

_mesh = plsc.VectorSubcoreMesh(core_axis_name="c", subcore_axis_name="s")


def _zero_fill_1d(ref, n):
    def body(i, _):
        ref[pl.ds(i * 16, 16)] = jnp.zeros((16,), jnp.float32)
        return 0
    lax.fori_loop(0, n // 16, body, 0)


def _hist_body(cpt, dst_hbm, out_hbm, idx_v, ones_v, zero_v, hist_sh):
    c = lax.axis_index("c")
    s = lax.axis_index("s")
    wid = s * NC + c

    def fill_ones(i, _):
        ones_v[pl.ds(i * 16, 16)] = jnp.ones((16,), jnp.float32)
        return 0
    lax.fori_loop(0, CHUNK // 16, fill_ones, 0)
    _zero_fill_1d(zero_v, RPT)
    pltpu.sync_copy(zero_v, hist_sh.at[pl.ds(s * RPT, RPT)])
    plsc.subcore_barrier()

    base = wid * cpt * CHUNK

    def body(t, _):
        off = base + t * CHUNK
        pltpu.sync_copy(dst_hbm.at[pl.ds(off, CHUNK)], idx_v)
        pltpu.sync_copy(ones_v, hist_sh.at[idx_v], add=True)
        return 0
    lax.fori_loop(0, cpt, body, 0)

    plsc.subcore_barrier()
    pltpu.sync_copy(hist_sh.at[pl.ds(s * RPT, RPT)],
                    out_hbm.at[pl.ds(c * N_PAD + s * RPT, RPT)])


def _acc_body(cpt, g_hbm, src_hbm, dst_hbm, out_hbm,
              src_v, dst_v, rows_v, zero_v, sem, acc_sh):
    c = lax.axis_index("c")
    s = lax.axis_index("s")
    wid = s * NC + c

    def fz(r, _):
        for q in range(D // 16):
            zero_v[r, pl.ds(q * 16, 16)] = jnp.zeros((16,), jnp.float32)
        return 0
    lax.fori_loop(0, CHUNK, fz, 0)
    for j in range(RPT // CHUNK):
        pltpu.sync_copy(zero_v, acc_sh.at[pl.ds(s * RPT + j * CHUNK, CHUNK)])
    plsc.subcore_barrier()

    base = wid * cpt * CHUNK

    def body(t, _):
        off = base + t * CHUNK
        pltpu.sync_copy(src_hbm.at[pl.ds(off, CHUNK)], src_v)
        pltpu.sync_copy(dst_hbm.at[pl.ds(off, CHUNK)], dst_v)
        pltpu.async_copy(g_hbm.at[src_v], rows_v, sem).wait()
        pltpu.sync_copy(rows_v, acc_sh.at[dst_v], add=True)
        return 0
    lax.fori_loop(0, cpt, body, 0)

    plsc.subcore_barrier()
    pltpu.sync_copy(acc_sh.at[pl.ds(s * RPT, RPT)],
                    out_hbm.at[pl.ds(c * N_PAD + s * RPT, RPT)])


def _make_hist(cpt):
    return pl.kernel(
        functools.partial(_hist_body, cpt),
        mesh=_mesh,
        out_type=jax.ShapeDtypeStruct((2 * N_PAD,), jnp.float32),
        scratch_types=[
            pltpu.VMEM((CHUNK,), jnp.int32),
            pltpu.VMEM((CHUNK,), jnp.float32),
            pltpu.VMEM((RPT,), jnp.float32),
            pltpu.VMEM_SHARED((N_PAD,), jnp.float32),
        ],
    )


def _make_acc(cpt):
    return pl.kernel(
        functools.partial(_acc_body, cpt),
        mesh=_mesh,
        out_type=jax.ShapeDtypeStruct((2 * N_PAD, D), jnp.float32),
        scratch_types=[
            pltpu.VMEM((CHUNK,), jnp.int32),
            pltpu.VMEM((CHUNK,), jnp.int32),
            pltpu.VMEM((CHUNK, D), jnp.float32),
            pltpu.VMEM((CHUNK, D), jnp.float32),
            pltpu.SemaphoreType.DMA,
            pltpu.VMEM_SHARED((N_PAD, D), jnp.float32),
        ],
    )


def _mm1_body(x_ref, w_ref, h0_ref, h1_ref, g_ref, ds_ref):
    i = pl.program_id(0)
    deg = 1.0 + h0_ref[...] + h1_ref[...]
    rows = lax.broadcasted_iota(jnp.int32, (BR, 1), 0) + i * BR
    ds = jnp.where(rows < N_NODES, lax.rsqrt(deg), 0.0)
    ds_ref[...] = ds
    g_ref[...] = jnp.dot(x_ref[...], w_ref[...],
                         preferred_element_type=jnp.float32) * ds


def _comb_mm_body(a0_ref, a1_ref, g_ref, ds_ref, b_ref, w_ref, out_ref):
    ds = ds_ref[...]
    z = ds * (a0_ref[...] + a1_ref[...] + g_ref[...]) + b_ref[...]
    z = jnp.maximum(z, 0.0)
    out_ref[...] = jnp.dot(z, w_ref[...],
                           preferred_element_type=jnp.float32) * ds


def _comb2_body(a0_ref, a1_ref, g_ref, ds_ref, b_ref, out_ref):
    out_ref[...] = (ds_ref[...] * (a0_ref[...] + a1_ref[...] + g_ref[...])
                    + b_ref[...])


def _row_spec(off):
    return pl.BlockSpec((BR, D), lambda i, o=off: (i + o, 0))


def _col_spec(off):
    return pl.BlockSpec((BR, 1), lambda i, o=off: (i + o, 0))


_full_w = pl.BlockSpec((D, D), lambda i: (0, 0))
_full_b = pl.BlockSpec((1, D), lambda i: (0, 0))


def _mm1(x_pad, W1, hist2):
    return pl.pallas_call(
        _mm1_body,
        grid=(GRID,),
        in_specs=[_row_spec(0), _full_w, _col_spec(0), _col_spec(GRID)],
        out_specs=[_row_spec(0), _col_spec(0)],
        out_shape=[jax.ShapeDtypeStruct((N_PAD, D), jnp.float32),
                   jax.ShapeDtypeStruct((N_PAD, 1), jnp.float32)],
    )(x_pad, W1, hist2, hist2)


def _comb_mm(acc2, g1, ds, b1, W2):
    return pl.pallas_call(
        _comb_mm_body,
        grid=(GRID,),
        in_specs=[_row_spec(0), _row_spec(GRID), _row_spec(0), _col_spec(0),
                  _full_b, _full_w],
        out_specs=_row_spec(0),
        out_shape=jax.ShapeDtypeStruct((N_PAD, D), jnp.float32),
    )(acc2, acc2, g1, ds, b1, W2)


def _comb2(acc2, g2, ds, b2):
    return pl.pallas_call(
        _comb2_body,
        grid=(GRID,),
        in_specs=[_row_spec(0), _row_spec(GRID), _row_spec(0), _col_spec(0),
                  _full_b],
        out_specs=_row_spec(0),
        out_shape=jax.ShapeDtypeStruct((N_PAD, D), jnp.float32),
    )(acc2, acc2, g2, ds, b2)


def kernel(x, edge_index, W1, b1, W2, b2):
    E = edge_index.shape[1]
    cpt = -(-E // (NW * CHUNK))
    e_pad = NW * cpt * CHUNK
    n_extra = N_PAD - N_NODES

    ei = edge_index.astype(jnp.int32)
    pad_rows = (N_NODES
                + jnp.arange(e_pad - E, dtype=jnp.int32) % n_extra)
    src = jnp.concatenate([ei[0], pad_rows])
    dst = jnp.concatenate([ei[1], pad_rows])

    x_pad = jnp.pad(x, ((0, n_extra), (0, 0)))
    b1r = b1.reshape(1, D)
    b2r = b2.reshape(1, D)

    hist = _make_hist(cpt)(dst)
    hist2 = hist.reshape(2 * N_PAD, 1)

    g1, ds = _mm1(x_pad, W1, hist2)
    acc1 = _make_acc(cpt)(g1, src, dst)
    g2 = _comb_mm(acc1, g1, ds, b1r, W2)
    acc2 = _make_acc(cpt)(g2, src, dst)
    out = _comb2(acc2, g2, ds, b2r)
    return out[:N_NODES]
